# baseline (speedup 1.0000x reference)
import jax
import jax.numpy as jnp
from jax.experimental import pallas as pl
from jax.experimental.pallas import tpu as pltpu

_NUM_CLASSES = 5


def _rank_loss_kernel(sca_ref, scb_ref, lca_ref, lcb_ref,
                      srow_ref, srev_ref, lrow_ref, lrev_ref,
                      out_ref, acc_ref):
    b = pl.program_id(0)
    rt = pl.program_id(1)
    nb = pl.num_programs(0)
    nt = pl.num_programs(1)
    tr = sca_ref.shape[0]
    g = lrow_ref.shape[2]

    @pl.when(jnp.logical_and(b == 0, rt == 0))
    def _init_totals():
        acc_ref[3] = 0.0
        acc_ref[4] = 0.0

    @pl.when(rt == 0)
    def _init_group():
        acc_ref[0] = 0.0
        lab = lrow_ref[0]
        s = srow_ref[0]
        sumsq = jnp.zeros((), jnp.float32)
        lin = jnp.zeros((), jnp.float32)
        for a in range(_NUM_CLASSES):
            cnt = jnp.sum(jnp.where(lab == a, 1.0, 0.0))
            sumsq += cnt * cnt
            lin += cnt * jnp.sum(s * jnp.sign(lab - a).astype(jnp.float32))
        acc_ref[1] = (float(g * g) - sumsq) * 0.5
        acc_ref[2] = lin

    sca = sca_ref[...]
    scb = scb_ref[...]
    lca = lca_ref[...]
    lcb = lcb_ref[...]
    srow = srow_ref[0]
    srev = srev_ref[0]
    lrow = lrow_ref[0]
    lrev = lrev_ref[0]

    h = g // 2

    def t_of(a):
        return jnp.log1p(jnp.exp(-a)) + 0.5 * a

    shape = (tr, h)
    r = rt * tr + jax.lax.broadcasted_iota(jnp.int32, shape, 0)
    j = jax.lax.broadcasted_iota(jnp.int32, shape, 1)
    top = j < r
    dl = jnp.where(top, sca - srow[:, :h], scb - srev[:, :h])
    ml = jnp.logical_or(jnp.logical_and(top, lca != lrow[:, :h]),
                        jnp.logical_and(jnp.logical_not(top),
                                        lcb != lrev[:, :h]))
    tot = jnp.sum(jnp.where(ml, t_of(jnp.abs(dl)), 0.0))
    dr = scb - srev[:, h:]
    mr = lcb != lrev[:, h:]
    tot += jnp.sum(jnp.where(mr, t_of(jnp.abs(dr)), 0.0))
    acc_ref[0] += tot

    @pl.when(rt == nt - 1)
    def _finalize_group():
        n_pairs = acc_ref[1]
        safe_n = jnp.where(n_pairs > 0, n_pairs, 1.0)
        loss = (acc_ref[0] - 0.5 * acc_ref[2]) / safe_n
        acc_ref[3] += jnp.where(n_pairs > 0, loss, 0.0)
        acc_ref[4] += jnp.where(n_pairs > 0, 1.0, 0.0)

        @pl.when(b == nb - 1)
        def _finalize_output():
            count = acc_ref[4]
            safe_c = jnp.where(count > 0, count, 1.0)
            out_ref[0, 0] = jnp.where(count > 0, acc_ref[3] / safe_c, 0.0)


def kernel(scores, labels, group_sizes):
    scores = scores.reshape(-1)
    labels = labels.reshape(-1)
    n = scores.shape[0]
    num_groups = group_sizes.shape[0]
    g = n // num_groups
    h = g // 2
    tr = 512
    nt = h // tr

    s2 = scores.reshape(num_groups, g)
    l2 = labels.reshape(num_groups, g)
    sca = s2[:, :h].reshape(num_groups * h, 1)
    scb = s2[:, :h - 1:-1].reshape(num_groups * h, 1)
    lca = l2[:, :h].reshape(num_groups * h, 1)
    lcb = l2[:, :h - 1:-1].reshape(num_groups * h, 1)
    srow = s2.reshape(num_groups, 1, g)
    srev = s2[:, ::-1].reshape(num_groups, 1, g)
    lrow = l2.reshape(num_groups, 1, g)
    lrev = l2[:, ::-1].reshape(num_groups, 1, g)

    col = pl.BlockSpec((tr, 1), lambda b, rt: (b * nt + rt, 0))
    row = pl.BlockSpec((1, 1, g), lambda b, rt: (b, 0, 0))

    out = pl.pallas_call(
        _rank_loss_kernel,
        grid=(num_groups, nt),
        in_specs=[col, col, col, col, row, row, row, row],
        out_specs=pl.BlockSpec(memory_space=pltpu.SMEM),
        out_shape=jax.ShapeDtypeStruct((1, 1), jnp.float32),
        scratch_shapes=[pltpu.SMEM((5,), jnp.float32)],
    )(sca, scb, lca, lcb, srow, srev, lrow, lrev)
    return out[0, 0]


# speedup vs baseline: 1.1392x; 1.1392x over previous
import jax
import jax.numpy as jnp
from jax.experimental import pallas as pl
from jax.experimental.pallas import tpu as pltpu

_NUM_CLASSES = 5


def _rank_loss_kernel(sca_ref, scb_ref, lca_ref, lcb_ref,
                      srow_ref, srev_ref, lrow_ref, lrev_ref,
                      out_ref, acc_ref):
    b = pl.program_id(0)
    rt = pl.program_id(1)
    nb = pl.num_programs(0)
    nt = pl.num_programs(1)
    tr = sca_ref.shape[0]
    g = lrow_ref.shape[2]

    @pl.when(jnp.logical_and(b == 0, rt == 0))
    def _init_totals():
        acc_ref[3] = 0.0
        acc_ref[4] = 0.0

    @pl.when(rt == 0)
    def _init_group():
        acc_ref[0] = 0.0
        lab = lrow_ref[0]
        s = srow_ref[0]
        sumsq = jnp.zeros((), jnp.float32)
        lin = jnp.zeros((), jnp.float32)
        for a in range(_NUM_CLASSES):
            cnt = jnp.sum(jnp.where(lab == a, 1.0, 0.0))
            sumsq += cnt * cnt
            lin += cnt * jnp.sum(s * jnp.sign(lab - a).astype(jnp.float32))
        acc_ref[1] = (float(g * g) - sumsq) * 0.5
        acc_ref[2] = lin

    sca = sca_ref[...]
    scb = scb_ref[...]
    lca = lca_ref[...]
    lcb = lcb_ref[...]
    srow = srow_ref[0]
    srev = srev_ref[0]
    lrow = lrow_ref[0]
    lrev = lrev_ref[0]

    h = g // 2
    c1 = -1.4426950408889634
    c2 = 0.5 / 0.6931471805599453

    def t_of(a):
        ab = a.astype(jnp.bfloat16)
        y = jnp.exp2(ab * jnp.bfloat16(c1))
        lg = jnp.log2(1.0 + y.astype(jnp.float32))
        return lg + c2 * a

    shape = (tr, h)
    r = rt * tr + jax.lax.broadcasted_iota(jnp.int32, shape, 0)
    j = jax.lax.broadcasted_iota(jnp.int32, shape, 1)
    top = j < r
    dl = jnp.where(top, sca - srow[:, :h], scb - srev[:, :h])
    ml = jnp.logical_or(jnp.logical_and(top, lca != lrow[:, :h]),
                        jnp.logical_and(jnp.logical_not(top),
                                        lcb != lrev[:, :h]))
    tot = jnp.sum(jnp.where(ml, t_of(jnp.abs(dl)), 0.0))
    dr = scb - srev[:, h:]
    mr = lcb != lrev[:, h:]
    tot += jnp.sum(jnp.where(mr, t_of(jnp.abs(dr)), 0.0))
    acc_ref[0] += tot

    @pl.when(rt == nt - 1)
    def _finalize_group():
        n_pairs = acc_ref[1]
        safe_n = jnp.where(n_pairs > 0, n_pairs, 1.0)
        ln2 = 0.6931471805599453
        loss = (ln2 * acc_ref[0] - 0.5 * acc_ref[2]) / safe_n
        acc_ref[3] += jnp.where(n_pairs > 0, loss, 0.0)
        acc_ref[4] += jnp.where(n_pairs > 0, 1.0, 0.0)

        @pl.when(b == nb - 1)
        def _finalize_output():
            count = acc_ref[4]
            safe_c = jnp.where(count > 0, count, 1.0)
            out_ref[0, 0] = jnp.where(count > 0, acc_ref[3] / safe_c, 0.0)


def kernel(scores, labels, group_sizes):
    scores = scores.reshape(-1)
    labels = labels.reshape(-1)
    n = scores.shape[0]
    num_groups = group_sizes.shape[0]
    g = n // num_groups
    h = g // 2
    tr = 512
    nt = h // tr

    s2 = scores.reshape(num_groups, g)
    l2 = labels.reshape(num_groups, g)
    sca = s2[:, :h].reshape(num_groups * h, 1)
    scb = s2[:, :h - 1:-1].reshape(num_groups * h, 1)
    lca = l2[:, :h].reshape(num_groups * h, 1)
    lcb = l2[:, :h - 1:-1].reshape(num_groups * h, 1)
    srow = s2.reshape(num_groups, 1, g)
    srev = s2[:, ::-1].reshape(num_groups, 1, g)
    lrow = l2.reshape(num_groups, 1, g)
    lrev = l2[:, ::-1].reshape(num_groups, 1, g)

    col = pl.BlockSpec((tr, 1), lambda b, rt: (b * nt + rt, 0))
    row = pl.BlockSpec((1, 1, g), lambda b, rt: (b, 0, 0))

    out = pl.pallas_call(
        _rank_loss_kernel,
        grid=(num_groups, nt),
        in_specs=[col, col, col, col, row, row, row, row],
        out_specs=pl.BlockSpec(memory_space=pltpu.SMEM),
        out_shape=jax.ShapeDtypeStruct((1, 1), jnp.float32),
        scratch_shapes=[pltpu.SMEM((5,), jnp.float32)],
    )(sca, scb, lca, lcb, srow, srev, lrow, lrev)
    return out[0, 0]


# speedup vs baseline: 1.1705x; 1.0275x over previous
import jax
import jax.numpy as jnp
from jax.experimental import pallas as pl
from jax.experimental.pallas import tpu as pltpu

_NUM_CLASSES = 5


def _rank_loss_kernel(sca_ref, scb_ref, lca_ref, lcb_ref,
                      srow_ref, srev_ref, lrow_ref, lrev_ref,
                      out_ref, acc_ref):
    b = pl.program_id(0)
    rt = pl.program_id(1)
    nb = pl.num_programs(0)
    nt = pl.num_programs(1)
    tr = sca_ref.shape[0]
    g = lrow_ref.shape[2]

    @pl.when(jnp.logical_and(b == 0, rt == 0))
    def _init_totals():
        acc_ref[3] = 0.0
        acc_ref[4] = 0.0

    @pl.when(rt == 0)
    def _init_group():
        acc_ref[0] = 0.0
        lab = lrow_ref[0]
        s = srow_ref[0]
        sumsq = jnp.zeros((), jnp.float32)
        lin = jnp.zeros((), jnp.float32)
        for a in range(_NUM_CLASSES):
            cnt = jnp.sum(jnp.where(lab == a, 1.0, 0.0))
            sumsq += cnt * cnt
            lin += cnt * jnp.sum(s * jnp.sign(lab - a).astype(jnp.float32))
        acc_ref[1] = (float(g * g) - sumsq) * 0.5
        acc_ref[2] = lin

    sca = sca_ref[...]
    scb = scb_ref[...]
    lca = lca_ref[...]
    lcb = lcb_ref[...]
    srow = srow_ref[0]
    srev = srev_ref[0]
    lrow = lrow_ref[0]
    lrev = lrev_ref[0]

    h = g // 2
    c1 = -1.4426950408889634
    c2 = 0.5 / 0.6931471805599453

    def t_of(a):
        return jnp.log2(1.0 + jnp.exp2(a * c1)) + c2 * a

    shape = (tr, h)
    r = rt * tr + jax.lax.broadcasted_iota(jnp.int32, shape, 0)
    j = jax.lax.broadcasted_iota(jnp.int32, shape, 1)
    top = j < r
    dl = jnp.where(top, sca - srow[:, :h], scb - srev[:, :h])
    ml = jnp.logical_or(jnp.logical_and(top, lca != lrow[:, :h]),
                        jnp.logical_and(jnp.logical_not(top),
                                        lcb != lrev[:, :h]))
    tot = jnp.sum(jnp.where(ml, t_of(jnp.abs(dl)), 0.0))
    dr = scb - srev[:, h:]
    mr = lcb != lrev[:, h:]
    tot += jnp.sum(jnp.where(mr, t_of(jnp.abs(dr)), 0.0))
    acc_ref[0] += tot

    @pl.when(rt == nt - 1)
    def _finalize_group():
        n_pairs = acc_ref[1]
        safe_n = jnp.where(n_pairs > 0, n_pairs, 1.0)
        ln2 = 0.6931471805599453
        loss = (ln2 * acc_ref[0] - 0.5 * acc_ref[2]) / safe_n
        acc_ref[3] += jnp.where(n_pairs > 0, loss, 0.0)
        acc_ref[4] += jnp.where(n_pairs > 0, 1.0, 0.0)

        @pl.when(b == nb - 1)
        def _finalize_output():
            count = acc_ref[4]
            safe_c = jnp.where(count > 0, count, 1.0)
            out_ref[0, 0] = jnp.where(count > 0, acc_ref[3] / safe_c, 0.0)


def kernel(scores, labels, group_sizes):
    scores = scores.reshape(-1)
    labels = labels.reshape(-1)
    n = scores.shape[0]
    num_groups = group_sizes.shape[0]
    g = n // num_groups
    h = g // 2
    tr = 512
    nt = h // tr

    s2 = scores.reshape(num_groups, g)
    l2 = labels.reshape(num_groups, g)
    sca = s2[:, :h].reshape(num_groups * h, 1)
    scb = s2[:, :h - 1:-1].reshape(num_groups * h, 1)
    lca = l2[:, :h].reshape(num_groups * h, 1)
    lcb = l2[:, :h - 1:-1].reshape(num_groups * h, 1)
    srow = s2.reshape(num_groups, 1, g)
    srev = s2[:, ::-1].reshape(num_groups, 1, g)
    lrow = l2.reshape(num_groups, 1, g)
    lrev = l2[:, ::-1].reshape(num_groups, 1, g)

    col = pl.BlockSpec((tr, 1), lambda b, rt: (b * nt + rt, 0))
    row = pl.BlockSpec((1, 1, g), lambda b, rt: (b, 0, 0))

    out = pl.pallas_call(
        _rank_loss_kernel,
        grid=(num_groups, nt),
        in_specs=[col, col, col, col, row, row, row, row],
        out_specs=pl.BlockSpec(memory_space=pltpu.SMEM),
        out_shape=jax.ShapeDtypeStruct((1, 1), jnp.float32),
        scratch_shapes=[pltpu.SMEM((5,), jnp.float32)],
    )(sca, scb, lca, lcb, srow, srev, lrow, lrev)
    return out[0, 0]


# speedup vs baseline: 1.1714x; 1.0008x over previous
import jax
import jax.numpy as jnp
from jax.experimental import pallas as pl
from jax.experimental.pallas import tpu as pltpu

_NUM_CLASSES = 5
_MIN_MIXED = 128


def _rank_loss_kernel(sca_ref, scb_ref, lca_ref, lcb_ref,
                      srow_ref, srev_ref, lrow_ref, lrev_ref,
                      out_ref, acc_ref):
    b = pl.program_id(0)
    nb = pl.num_programs(0)
    g = lrow_ref.shape[2]
    h = g // 2
    c1 = -1.4426950408889634
    c2 = 0.5 / 0.6931471805599453
    ln2 = 0.6931471805599453

    @pl.when(b == 0)
    def _init_totals():
        acc_ref[0] = 0.0
        acc_ref[1] = 0.0

    lab = lrow_ref[0]
    s = srow_ref[0]
    sumsq = jnp.zeros((), jnp.float32)
    lin = jnp.zeros((), jnp.float32)
    for a in range(_NUM_CLASSES):
        cnt = jnp.sum(jnp.where(lab == a, 1.0, 0.0))
        sumsq += cnt * cnt
        lin += cnt * jnp.sum(s * jnp.sign(lab - a).astype(jnp.float32))
    n_pairs = (float(g * g) - sumsq) * 0.5

    sca = sca_ref[...]
    scb = scb_ref[...]
    lca = lca_ref[...]
    lcb = lcb_ref[...]
    srow = s
    srev = srev_ref[0]
    lrow = lab
    lrev = lrev_ref[0]

    def t_of(a):
        return jnp.log2(1.0 + jnp.exp2(a * c1)) + c2 * a

    sums = []

    def emit_top(r0, r1, c0, c1):
        d = sca[r0:r1] - srow[:, c0:c1]
        m = lca[r0:r1] != lrow[:, c0:c1]
        sums.append(jnp.sum(jnp.where(m, t_of(jnp.abs(d)), 0.0)))

    def emit_bottom(r0, r1, c0, c1):
        d = scb[r0:r1] - srev[:, c0:c1]
        m = lcb[r0:r1] != lrev[:, c0:c1]
        sums.append(jnp.sum(jnp.where(m, t_of(jnp.abs(d)), 0.0)))

    def emit_mixed(r0, r1, c0, c1):
        shp = (r1 - r0, c1 - c0)
        rr = r0 + jax.lax.broadcasted_iota(jnp.int32, shp, 0)
        jj = c0 + jax.lax.broadcasted_iota(jnp.int32, shp, 1)
        top = jj < rr
        d = jnp.where(top, sca[r0:r1] - srow[:, c0:c1],
                      scb[r0:r1] - srev[:, c0:c1])
        m = jnp.logical_or(
            jnp.logical_and(top, lca[r0:r1] != lrow[:, c0:c1]),
            jnp.logical_and(jnp.logical_not(top),
                            lcb[r0:r1] != lrev[:, c0:c1]))
        sums.append(jnp.sum(jnp.where(m, t_of(jnp.abs(d)), 0.0)))

    def emit(r0, r1, c0, c1):
        if c1 <= r0:
            emit_top(r0, r1, c0, c1)
        elif c0 >= r1 - 1:
            emit_bottom(r0, r1, c0, c1)
        elif r1 - r0 <= _MIN_MIXED:
            emit_mixed(r0, r1, c0, c1)
        else:
            rm = (r0 + r1) // 2
            cm = (c0 + c1) // 2
            emit(r0, rm, c0, cm)
            emit(r0, rm, cm, c1)
            emit(rm, r1, c0, cm)
            emit(rm, r1, cm, c1)

    emit(0, h, 0, h)
    emit(0, h, h, g)
    tot = sums[0]
    for s_part in sums[1:]:
        tot = tot + s_part

    safe_n = jnp.where(n_pairs > 0, n_pairs, 1.0)
    loss = (ln2 * tot - 0.5 * lin) / safe_n
    acc_ref[0] += jnp.where(n_pairs > 0, loss, 0.0)
    acc_ref[1] += jnp.where(n_pairs > 0, 1.0, 0.0)

    @pl.when(b == nb - 1)
    def _finalize_output():
        count = acc_ref[1]
        safe_c = jnp.where(count > 0, count, 1.0)
        out_ref[0, 0] = jnp.where(count > 0, acc_ref[0] / safe_c, 0.0)


def kernel(scores, labels, group_sizes):
    scores = scores.reshape(-1)
    labels = labels.reshape(-1)
    n = scores.shape[0]
    num_groups = group_sizes.shape[0]
    g = n // num_groups
    h = g // 2

    s2 = scores.reshape(num_groups, g)
    l2 = labels.reshape(num_groups, g)
    sca = s2[:, :h].reshape(num_groups * h, 1)
    scb = s2[:, :h - 1:-1].reshape(num_groups * h, 1)
    lca = l2[:, :h].reshape(num_groups * h, 1)
    lcb = l2[:, :h - 1:-1].reshape(num_groups * h, 1)
    srow = s2.reshape(num_groups, 1, g)
    srev = s2[:, ::-1].reshape(num_groups, 1, g)
    lrow = l2.reshape(num_groups, 1, g)
    lrev = l2[:, ::-1].reshape(num_groups, 1, g)

    col = pl.BlockSpec((h, 1), lambda b: (b, 0))
    row = pl.BlockSpec((1, 1, g), lambda b: (b, 0, 0))

    out = pl.pallas_call(
        _rank_loss_kernel,
        grid=(num_groups,),
        in_specs=[col, col, col, col, row, row, row, row],
        out_specs=pl.BlockSpec(memory_space=pltpu.SMEM),
        out_shape=jax.ShapeDtypeStruct((1, 1), jnp.float32),
        scratch_shapes=[pltpu.SMEM((2,), jnp.float32)],
    )(sca, scb, lca, lcb, srow, srev, lrow, lrev)
    return out[0, 0]
